# baseline (speedup 1.0000x reference)
import functools

import jax
import jax.numpy as jnp
from jax import lax
from jax.experimental import pallas as pl
from jax.experimental.pallas import tpu as pltpu
from jax.experimental.pallas import tpu_sc as plsc

B, L = 16384, 16
V, D = 100000, 768

NC, NS = 2, 16
NW = NC * NS
B_PER_W = B // NW
CHUNK = 32
NCHUNK = B_PER_W // CHUNK
NBUF = 4
INFLIGHT = 2
BIAS_W = 128
BIAS_ROWS = (V + BIAS_W - 1) // BIAS_W

_mesh = plsc.VectorSubcoreMesh(core_axis_name="c", subcore_axis_name="s")


@functools.partial(
    pl.kernel,
    mesh=_mesh,
    compiler_params=pltpu.CompilerParams(needs_layout_passes=False),
    out_type=(
        jax.ShapeDtypeStruct((B, D), jnp.float32),
        jax.ShapeDtypeStruct((B,), jnp.float32),
    ),
    scratch_types=[
        pltpu.VMEM((B_PER_W * L // 128, 128), jnp.int32),
        pltpu.VMEM((NCHUNK, CHUNK), jnp.int32),
        pltpu.VMEM((NCHUNK, CHUNK), jnp.int32),
        pltpu.VMEM((NBUF, CHUNK, D), jnp.float32),
        pltpu.VMEM((2, CHUNK, BIAS_W), jnp.float32),
        pltpu.VMEM((B_PER_W,), jnp.float32),
        [pltpu.SemaphoreType.DMA] * NBUF,
        [pltpu.SemaphoreType.DMA] * NBUF,
        [pltpu.SemaphoreType.DMA] * 2,
    ],
)
def _gather_kernel(tsf_hbm, table_hbm, bias_hbm, out_hbm,
                   bias_out_hbm, tsf_v, idx_v, rid_v, rows_v, brows_v, bias_v,
                   gsems, wsems, bsems):
    wid = lax.axis_index("s") * NC + lax.axis_index("c")
    base = wid * B_PER_W

    pltpu.sync_copy(tsf_hbm.at[wid], tsf_v)
    lanes = lax.iota(jnp.int32, 16)
    rows0 = lanes >> 3
    cols = (lanes & 7) << 4
    for j in range(NCHUNK):
        for t in range(CHUNK // 16):
            g16 = j * (CHUNK // 16) + t
            ids16 = plsc.load_gather(tsf_v, [rows0 + g16 * 2, cols])
            idx_v[j, pl.ds(t * 16, 16)] = ids16
            rid_v[j, pl.ds(t * 16, 16)] = ids16 >> 7

    def start_gather(j):
        return pltpu.async_copy(
            table_hbm.at[idx_v.at[j]], rows_v.at[j % NBUF], gsems[j % NBUF])

    def start_write(j):
        return pltpu.async_copy(
            rows_v.at[j % NBUF],
            out_hbm.at[pl.ds(base + j * CHUNK, CHUNK)], wsems[j % NBUF])

    def start_bias(j):
        return pltpu.async_copy(
            bias_hbm.at[rid_v.at[j]], brows_v.at[j % 2], bsems[j % 2])

    g = {j: start_gather(j) for j in range(INFLIGHT)}
    bg = {j: start_bias(j) for j in range(2)}
    w = {}
    for j in range(NCHUNK):
        g[j].wait()
        w[j] = start_write(j)
        k = j + INFLIGHT
        if k < NCHUNK:
            if k - NBUF >= 0:
                w[k - NBUF].wait()
            g[k] = start_gather(k)
        bg[j].wait()
        for t in range(CHUNK // 16):
            ids16 = idx_v[j, pl.ds(t * 16, 16)]
            offs = ids16 & (BIAS_W - 1)
            rows = lax.iota(jnp.int32, 16) + (t * 16)
            bias_v[pl.ds(j * CHUNK + t * 16, 16)] = plsc.load_gather(
                brows_v.at[j % 2], [rows, offs])
        if j + 2 < NCHUNK:
            bg[j + 2] = start_bias(j + 2)
    waited = {k - NBUF for k in range(INFLIGHT, NCHUNK) if k - NBUF >= 0}
    for j in range(NCHUNK):
        if j not in waited:
            w[j].wait()

    pltpu.sync_copy(bias_v, bias_out_hbm.at[pl.ds(base, B_PER_W)])


def kernel(target_surface_forms, target_priors, input_embeddings, bias):
    del target_priors
    tsf = target_surface_forms.astype(jnp.int32).reshape(
        NW, B_PER_W * L // 128, 128)
    bias2d = jnp.pad(bias[:, 0], (0, BIAS_ROWS * BIAS_W - V))
    bias2d = bias2d.reshape(BIAS_ROWS, BIAS_W)
    emb, b = _gather_kernel(tsf, input_embeddings, bias2d)
    return emb, b


# speedup vs baseline: 1.1590x; 1.1590x over previous
import functools

import jax
import jax.numpy as jnp
from jax import lax
from jax.experimental import pallas as pl
from jax.experimental.pallas import tpu as pltpu
from jax.experimental.pallas import tpu_sc as plsc

B, L = 16384, 16
V, D = 100000, 768

NC, NS = 2, 16
NW = NC * NS
B_PER_W = B // NW
CHUNK = 32
NCHUNK = B_PER_W // CHUNK
NBUF = 4
INFLIGHT = 3
BIAS_W = 128
BIAS_ROWS = (V + BIAS_W - 1) // BIAS_W

_mesh = plsc.VectorSubcoreMesh(core_axis_name="c", subcore_axis_name="s")


@functools.partial(
    pl.kernel,
    mesh=_mesh,
    compiler_params=pltpu.CompilerParams(needs_layout_passes=False),
    out_type=(
        jax.ShapeDtypeStruct((B, D), jnp.float32),
        jax.ShapeDtypeStruct((B,), jnp.float32),
    ),
    scratch_types=[
        pltpu.VMEM((NCHUNK, CHUNK), jnp.int32),
        pltpu.VMEM((NCHUNK, CHUNK), jnp.int32),
        pltpu.VMEM((NBUF, CHUNK, D), jnp.float32),
        pltpu.VMEM((2, CHUNK, BIAS_W), jnp.float32),
        pltpu.VMEM((B_PER_W,), jnp.float32),
        [pltpu.SemaphoreType.DMA] * NBUF,
        [pltpu.SemaphoreType.DMA] * NBUF,
        [pltpu.SemaphoreType.DMA] * 2,
    ],
)
def _gather_kernel(ids_hbm, rid_hbm, table_hbm, bias_hbm, out_hbm,
                   bias_out_hbm, idx_v, rid_v, rows_v, brows_v, bias_v,
                   gsems, wsems, bsems):
    wid = lax.axis_index("s") * NC + lax.axis_index("c")
    base = wid * B_PER_W

    pltpu.sync_copy(ids_hbm.at[wid], idx_v)
    pltpu.sync_copy(rid_hbm.at[wid], rid_v)

    def start_gather(j):
        return pltpu.async_copy(
            table_hbm.at[idx_v.at[j]], rows_v.at[j % NBUF], gsems[j % NBUF])

    def start_write(j):
        return pltpu.async_copy(
            rows_v.at[j % NBUF],
            out_hbm.at[pl.ds(base + j * CHUNK, CHUNK)], wsems[j % NBUF])

    def start_bias(j):
        return pltpu.async_copy(
            bias_hbm.at[rid_v.at[j]], brows_v.at[j % 2], bsems[j % 2])

    g = {j: start_gather(j) for j in range(INFLIGHT)}
    bg = {j: start_bias(j) for j in range(2)}
    w = {}
    for j in range(NCHUNK):
        g[j].wait()
        w[j] = start_write(j)
        k = j + INFLIGHT
        if k < NCHUNK:
            if k - NBUF >= 0:
                w[k - NBUF].wait()
            g[k] = start_gather(k)
        bg[j].wait()
        for t in range(CHUNK // 16):
            ids16 = idx_v[j, pl.ds(t * 16, 16)]
            offs = ids16 & (BIAS_W - 1)
            rows = lax.iota(jnp.int32, 16) + (t * 16)
            bias_v[pl.ds(j * CHUNK + t * 16, 16)] = plsc.load_gather(
                brows_v.at[j % 2], [rows, offs])
        if j + 2 < NCHUNK:
            bg[j + 2] = start_bias(j + 2)
    waited = {k - NBUF for k in range(INFLIGHT, NCHUNK) if k - NBUF >= 0}
    for j in range(NCHUNK):
        if j not in waited:
            w[j].wait()

    pltpu.sync_copy(bias_v, bias_out_hbm.at[pl.ds(base, B_PER_W)])


def kernel(target_surface_forms, target_priors, input_embeddings, bias):
    del target_priors
    ids = target_surface_forms[:, 0].astype(jnp.int32)
    rid = (ids >> 7).reshape(NW, NCHUNK, CHUNK)
    ids = ids.reshape(NW, NCHUNK, CHUNK)
    bias2d = jnp.pad(bias[:, 0], (0, BIAS_ROWS * BIAS_W - V))
    bias2d = bias2d.reshape(BIAS_ROWS, BIAS_W)
    emb, b = _gather_kernel(ids, rid, input_embeddings, bias2d)
    return emb, b


# speedup vs baseline: 1.2291x; 1.0605x over previous
import functools

import jax
import jax.numpy as jnp
from jax import lax
from jax.experimental import pallas as pl
from jax.experimental.pallas import tpu as pltpu
from jax.experimental.pallas import tpu_sc as plsc

B, L = 16384, 16
V, D = 100000, 768

NC, NS = 2, 16
NW = NC * NS
B_PER_W = B // NW
CHUNK = 32
NCHUNK = B_PER_W // CHUNK
NBUF = 4
INFLIGHT = 2

_mesh = plsc.VectorSubcoreMesh(core_axis_name="c", subcore_axis_name="s")


@functools.partial(
    pl.kernel,
    mesh=_mesh,
    compiler_params=pltpu.CompilerParams(needs_layout_passes=False),
    out_type=(
        jax.ShapeDtypeStruct((B, D), jnp.float32),
        jax.ShapeDtypeStruct((B,), jnp.float32),
    ),
    scratch_types=[
        pltpu.VMEM((NCHUNK, CHUNK), jnp.int32),
        pltpu.VMEM((NBUF, CHUNK, D), jnp.float32),
        pltpu.VMEM((B_PER_W,), jnp.float32),
        pltpu.VMEM_SHARED((V,), jnp.float32),
        [pltpu.SemaphoreType.DMA] * NBUF,
        [pltpu.SemaphoreType.DMA] * NBUF,
        pltpu.SemaphoreType.DMA,
    ],
)
def _gather_kernel(ids_hbm, table_hbm, bias_hbm, out_hbm,
                   bias_out_hbm, idx_v, rows_v, bias_v, bias_sp,
                   gsems, wsems, semb):
    sid = lax.axis_index("s")
    wid = sid * NC + lax.axis_index("c")
    base = wid * B_PER_W

    pltpu.sync_copy(ids_hbm.at[wid], idx_v)

    def start_gather(j):
        return pltpu.async_copy(
            table_hbm.at[idx_v.at[j]], rows_v.at[j % NBUF], gsems[j % NBUF])

    def start_write(j):
        return pltpu.async_copy(
            rows_v.at[j % NBUF],
            out_hbm.at[pl.ds(base + j * CHUNK, CHUNK)], wsems[j % NBUF])

    g = {j: start_gather(j) for j in range(INFLIGHT)}

    @pl.when(sid == 0)
    def _():
        pltpu.sync_copy(bias_hbm, bias_sp)
    plsc.subcore_barrier()

    bias_copies = [
        pltpu.async_copy(bias_sp.at[idx_v.at[j]],
                         bias_v.at[pl.ds(j * CHUNK, CHUNK)], semb)
        for j in range(NCHUNK)
    ]

    w = {}
    for j in range(NCHUNK):
        g[j].wait()
        w[j] = start_write(j)
        k = j + INFLIGHT
        if k < NCHUNK:
            if k - NBUF >= 0:
                w[k - NBUF].wait()
            g[k] = start_gather(k)
    waited = {k - NBUF for k in range(INFLIGHT, NCHUNK) if k - NBUF >= 0}
    for j in range(NCHUNK):
        if j not in waited:
            w[j].wait()

    for c in bias_copies:
        c.wait()
    pltpu.sync_copy(bias_v, bias_out_hbm.at[pl.ds(base, B_PER_W)])


def kernel(target_surface_forms, target_priors, input_embeddings, bias):
    del target_priors
    ids = target_surface_forms[:, 0].astype(jnp.int32)
    ids = ids.reshape(NW, NCHUNK, CHUNK)
    emb, b = _gather_kernel(ids, input_embeddings, bias[:, 0])
    return emb, b
